# baseline (speedup 1.0000x reference)
import functools
import math

import jax
import jax.numpy as jnp
from jax import lax
from jax.experimental import pallas as pl
from jax.experimental.pallas import tpu as pltpu
from jax.experimental.pallas import tpu_sc as plsc

_N = 8388608
_NW = 32
_CHUNK = _N // _NW


def _val_vec(tv):
    x = 0.001571 * (tv - 500.0)
    u = x - (math.pi / 2.0)
    u2 = u * u
    c = 1.0 + u2 * (-1.0 / 2 + u2 * (1.0 / 24 + u2 * (-1.0 / 720 + u2 * (
        1.0 / 40320 + u2 * (-1.0 / 3628800 + u2 * (1.0 / 479001600))))))
    cond = (tv > 500.0) & (tv < 2502.54614894971)
    return 14.625 * jnp.where(cond, 0.01 * c, 0.0)


@functools.partial(
    pl.kernel,
    mesh=plsc.VectorSubcoreMesh(core_axis_name="c", subcore_axis_name="s"),
    out_type=jax.ShapeDtypeStruct((_N,), jnp.float32),
    scratch_types=[
        pltpu.VMEM((16,), jnp.float32),
        pltpu.VMEM((16,), jnp.float32),
    ],
)
def _sc_assign(t_hbm, w_hbm, out_hbm, tv_buf, pv_buf):
    wid = lax.axis_index("s") * 2 + lax.axis_index("c")
    base = wid * _CHUNK
    pltpu.sync_copy(w_hbm.at[pl.ds(base, _CHUNK)], out_hbm.at[pl.ds(base, _CHUNK)])

    @pl.when(wid == 0)
    def _():
        tv_buf[...] = jnp.zeros((16,), jnp.float32)
        pltpu.sync_copy(t_hbm, tv_buf.at[pl.ds(0, 1)])
        pltpu.sync_copy(w_hbm.at[pl.ds(0, 16)], pv_buf)
        val = _val_vec(tv_buf[...])
        lane = lax.iota(jnp.int32, 16)
        pv_buf[...] = jnp.where(lane == 0, val, pv_buf[...])
        pltpu.sync_copy(pv_buf, out_hbm.at[pl.ds(0, 16)])


def kernel(y, w, c, t):
    return _sc_assign(t.reshape(1), w)


# speedup vs baseline: 23.6425x; 23.6425x over previous
import functools
import math

import jax
import jax.numpy as jnp
from jax import lax
from jax.experimental import pallas as pl
from jax.experimental.pallas import tpu as pltpu
from jax.experimental.pallas import tpu_sc as plsc

_N = 8388608
_NW = 32
_CHUNK = _N // _NW
_SUB = 32768
_NSUB = _CHUNK // _SUB


def _val_vec(tv):
    x = 0.001571 * (tv - 500.0)
    u = x - (math.pi / 2.0)
    u2 = u * u
    c = 1.0 + u2 * (-1.0 / 2 + u2 * (1.0 / 24 + u2 * (-1.0 / 720 + u2 * (
        1.0 / 40320 + u2 * (-1.0 / 3628800 + u2 * (1.0 / 479001600))))))
    cond = (tv > 500.0) & (tv < 2502.54614894971)
    return 14.625 * jnp.where(cond, 0.01 * c, 0.0)


@functools.partial(
    pl.kernel,
    mesh=plsc.VectorSubcoreMesh(core_axis_name="c", subcore_axis_name="s"),
    out_type=jax.ShapeDtypeStruct((_N,), jnp.float32),
    scratch_types=[
        pltpu.VMEM((16,), jnp.float32),
        pltpu.VMEM((2, _SUB), jnp.float32),
        pltpu.SemaphoreType.DMA,
        pltpu.SemaphoreType.DMA,
    ],
)
def _sc_assign(t_hbm, w_hbm, out_hbm, tv_buf, stage, sem0, sem1):
    wid = lax.axis_index("s") * 2 + lax.axis_index("c")
    base = wid * _CHUNK
    sems = (sem0, sem1)

    copies = [None, None]
    copies[0] = pltpu.async_copy(w_hbm.at[pl.ds(base, _SUB)], stage.at[0], sems[0])
    for i in range(_NSUB):
        cur = i % 2
        copies[cur].wait()
        if i + 1 < _NSUB:
            nxt = (i + 1) % 2
            copies[nxt] = pltpu.async_copy(
                w_hbm.at[pl.ds(base + (i + 1) * _SUB, _SUB)], stage.at[nxt], sems[nxt])
        if i == 0:
            @pl.when(wid == 0)
            def _():
                tv_buf[...] = jnp.zeros((16,), jnp.float32)
                pltpu.sync_copy(t_hbm, tv_buf.at[pl.ds(0, 1)])
                val = _val_vec(tv_buf[...])
                lane = lax.iota(jnp.int32, 16)
                head = stage.at[0]
                head[pl.ds(0, 16)] = jnp.where(lane == 0, val, head[pl.ds(0, 16)])
        pltpu.sync_copy(stage.at[cur], out_hbm.at[pl.ds(base + i * _SUB, _SUB)])


def kernel(y, w, c, t):
    return _sc_assign(t.reshape(1), w)
